# baseline (speedup 1.0000x reference)
import functools

import jax
import jax.numpy as jnp
from jax import lax
from jax.experimental import pallas as pl
from jax.experimental.pallas import tpu as pltpu
from jax.experimental.pallas import tpu_sc as plsc

K = 4

_NC = 2
_NS = 16
_NW = _NC * _NS

_C_SC = 32
_SC_OUT = 80
_NBUF = 4
_LOOKAHEAD = 2


def _stats_kernel(x_ref, sum_ref, topk_ref):
    v = x_ref[...]
    total = jnp.sum(v, axis=(1, 2))
    sum_ref[...] = total[:, None]

    cb, r, lanes = v.shape
    g = r // 8

    def body(i, carry):
        a1, a2, a3, a4 = carry
        s = x_ref[:, pl.ds(i * 8, 8), :]
        t = jnp.maximum(a1, s); s = jnp.minimum(a1, s); a1 = t
        t = jnp.maximum(a2, s); s = jnp.minimum(a2, s); a2 = t
        t = jnp.maximum(a3, s); s = jnp.minimum(a3, s); a3 = t
        a4 = jnp.maximum(a4, s)
        return a1, a2, a3, a4

    neg = jnp.full((cb, 8, lanes), -jnp.inf, jnp.float32)
    a1, a2, a3, a4 = jax.lax.fori_loop(0, g, body, (neg, neg, neg, neg))
    cand = jnp.concatenate([a1, a2, a3, a4], axis=1)

    acc = jnp.zeros((cb,), jnp.float32)
    k_rem = jnp.full((cb,), float(K))
    for _ in range(K):
        m = jnp.max(cand, axis=(1, 2))
        eq = cand == m[:, None, None]
        cnt = jnp.sum(eq.astype(jnp.float32), axis=(1, 2))
        take = jnp.minimum(cnt, k_rem)
        acc = acc + jnp.where(take > 0, m * take, 0.0)
        k_rem = k_rem - take
        cand = jnp.where(eq, -jnp.inf, cand)
    topk_ref[...] = acc[:, None]


def _sc_stats_body(x_hbm, out_hbm, b0, b1, b2, b3, s0, s1, s2, s3, ov,
                   *, n, ch0, chunk):
    w = lax.axis_index("s") * _NC + lax.axis_index("c")
    nchunk = n // chunk
    nv = chunk // 256
    bufs = [b0, b1, b2, b3]
    sems = [s0, s1, s2, s3]

    def off(t):
        return (ch0 + w) * n + t * chunk

    hin = {}
    for t in range(min(_LOOKAHEAD, nchunk)):
        hin[t] = pltpu.async_copy(
            x_hbm.at[pl.ds(off(t), chunk)], bufs[t % _NBUF], sems[t % _NBUF])

    neg = jnp.full((16,), -jnp.inf, jnp.float32)
    carry = (neg, neg, neg, neg, jnp.zeros((16,), jnp.float32))
    for t in range(nchunk):
        hin[t].wait()
        nt = t + _LOOKAHEAD
        if nt < nchunk:
            hin[nt] = pltpu.async_copy(
                x_hbm.at[pl.ds(off(nt), chunk)],
                bufs[nt % _NBUF], sems[nt % _NBUF])
        buf = bufs[t % _NBUF]

        def vbody(i, cr, buf=buf):
            a1, a2, a3, a4, ps = cr
            base = i * 256
            for j in range(16):
                s = buf[pl.ds(base + j * 16, 16)]
                ps = ps + s
                t_ = jnp.maximum(a1, s); s = jnp.minimum(a1, s); a1 = t_
                t_ = jnp.maximum(a2, s); s = jnp.minimum(a2, s); a2 = t_
                t_ = jnp.maximum(a3, s); s = jnp.minimum(a3, s); a3 = t_
                a4 = jnp.maximum(a4, s)
            return a1, a2, a3, a4, ps

        carry = lax.fori_loop(0, nv, vbody, carry)

    a1, a2, a3, a4, ps = carry
    ov[pl.ds(0, 16)] = a1
    ov[pl.ds(16, 16)] = a2
    ov[pl.ds(32, 16)] = a3
    ov[pl.ds(48, 16)] = a4
    ov[pl.ds(64, 16)] = ps
    pltpu.sync_copy(ov, out_hbm.at[pl.ds(w * _SC_OUT, _SC_OUT)])


def _gate_kernel(sum_lo_ref, topk_lo_ref, sc_ref, w1_ref, b1_ref, w2_ref,
                 b2_ref, gate_ref, *, inv_n):
    sc = sc_ref[...]
    ps = jnp.sum(sc[:, 64:80], axis=1)
    cand = sc[:, 0:64]

    acc = jnp.zeros((_C_SC,), jnp.float32)
    k_rem = jnp.full((_C_SC,), float(K))
    for _ in range(K):
        m = jnp.max(cand, axis=1)
        eq = cand == m[:, None]
        cnt = jnp.sum(eq.astype(jnp.float32), axis=1)
        take = jnp.minimum(cnt, k_rem)
        acc = acc + jnp.where(take > 0, m * take, 0.0)
        k_rem = k_rem - take
        cand = jnp.where(eq, -jnp.inf, cand)

    avg = jnp.concatenate([sum_lo_ref[...][:, 0], ps], axis=0) * inv_n
    tk = jnp.concatenate([topk_lo_ref[...][:, 0], acc], axis=0)

    def fc(v):
        h = jnp.dot(w1_ref[...], v, preferred_element_type=jnp.float32)
        h = jnp.maximum(h + b1_ref[...], 0.0)
        o = jnp.dot(w2_ref[...], h, preferred_element_type=jnp.float32)
        return o + b2_ref[...]

    gate_ref[...] = jax.nn.sigmoid(fc(avg[:, None]) + fc(tk[:, None]))


def _scale_kernel(x_ref, gate_ref, y_ref):
    y_ref[...] = x_ref[...] * gate_ref[...][:, :, None]


def kernel(x, W1, b1, W2, b2):
    b, c, d, h, w = x.shape
    n = d * h * w
    assert b == 1
    lanes = 128
    rows = n // lanes
    xr = x.reshape(c, rows, lanes)
    c_lo = c - _C_SC

    cb = 8
    sums_lo, topks_lo = pl.pallas_call(
        _stats_kernel,
        grid=(c_lo // cb,),
        in_specs=[pl.BlockSpec((cb, rows, lanes), lambda i: (i, 0, 0))],
        out_specs=[
            pl.BlockSpec((cb, 1), lambda i: (i, 0)),
            pl.BlockSpec((cb, 1), lambda i: (i, 0)),
        ],
        out_shape=[
            jax.ShapeDtypeStruct((c_lo, 1), jnp.float32),
            jax.ShapeDtypeStruct((c_lo, 1), jnp.float32),
        ],
        compiler_params=pltpu.CompilerParams(
            dimension_semantics=("parallel",)),
    )(xr)

    chunk = 25088
    assert n % chunk == 0
    sc_stats = pl.kernel(
        functools.partial(_sc_stats_body, n=n, ch0=c_lo, chunk=chunk),
        out_type=jax.ShapeDtypeStruct((_C_SC * _SC_OUT,), jnp.float32),
        mesh=plsc.VectorSubcoreMesh(core_axis_name="c", subcore_axis_name="s"),
        scratch_types=(
            [pltpu.VMEM((chunk,), jnp.float32) for _ in range(_NBUF)]
            + [pltpu.SemaphoreType.DMA for _ in range(_NBUF)]
            + [pltpu.VMEM((_SC_OUT,), jnp.float32)]
        ),
    )(x.reshape(c * n))

    gate = pl.pallas_call(
        functools.partial(_gate_kernel, inv_n=1.0 / n),
        out_shape=jax.ShapeDtypeStruct((c, 1), jnp.float32),
    )(sums_lo, topks_lo, sc_stats.reshape(_C_SC, _SC_OUT),
      W1, b1[:, None], W2, b2[:, None])

    y = pl.pallas_call(
        _scale_kernel,
        grid=(c // cb,),
        in_specs=[
            pl.BlockSpec((cb, rows, lanes), lambda i: (i, 0, 0)),
            pl.BlockSpec((cb, 1), lambda i: (i, 0)),
        ],
        out_specs=pl.BlockSpec((cb, rows, lanes), lambda i: (i, 0, 0)),
        out_shape=jax.ShapeDtypeStruct((c, rows, lanes), jnp.float32),
        compiler_params=pltpu.CompilerParams(
            dimension_semantics=("parallel",)),
    )(xr, gate)

    out = gate.reshape(b, c, 1, 1, 1)
    return (y.reshape(b, c, d, h, w), out)


# speedup vs baseline: 1.3511x; 1.3511x over previous
import functools

import jax
import jax.numpy as jnp
from jax.experimental import pallas as pl
from jax.experimental.pallas import tpu as pltpu

K = 4
_NBUF = 4
_NOB = 4


def _stats_block(buf, sum_ref, topk_ref, t, cb):
    v = buf[...]
    total = jnp.sum(v, axis=(1, 2))
    sum_ref[pl.ds(t * cb, cb), :] = total[:, None]

    _, rows, lanes = v.shape
    g = rows // 8

    def body(s_i, carry):
        a1, a2, a3, a4 = carry
        s = buf[:, pl.ds(s_i * 8, 8), :]
        t_ = jnp.maximum(a1, s); s = jnp.minimum(a1, s); a1 = t_
        t_ = jnp.maximum(a2, s); s = jnp.minimum(a2, s); a2 = t_
        t_ = jnp.maximum(a3, s); s = jnp.minimum(a3, s); a3 = t_
        a4 = jnp.maximum(a4, s)
        return a1, a2, a3, a4

    neg = jnp.full((cb, 8, lanes), -jnp.inf, jnp.float32)
    a1, a2, a3, a4 = jax.lax.fori_loop(0, g, body, (neg, neg, neg, neg))
    cand = jnp.concatenate([a1, a2, a3, a4], axis=1)

    acc = jnp.zeros((cb,), jnp.float32)
    k_rem = jnp.full((cb,), float(K))
    for _ in range(K):
        m = jnp.max(cand, axis=(1, 2))
        eq = cand == m[:, None, None]
        cnt = jnp.sum(eq.astype(jnp.float32), axis=(1, 2))
        take = jnp.minimum(cnt, k_rem)
        acc = acc + jnp.where(take > 0, m * take, 0.0)
        k_rem = k_rem - take
        cand = jnp.where(eq, -jnp.inf, cand)
    topk_ref[pl.ds(t * cb, cb), :] = acc[:, None]


def _ring_kernel(x_hbm, w1_ref, b1_ref, w2_ref, b2_ref, y_hbm, gate_ref,
                 *refs, nblk, cb, inv_n):
    bufs = refs[:_NBUF]
    obufs = refs[_NBUF:_NBUF + _NOB]
    sum_ref, topk_ref, gatev_ref = refs[_NBUF + _NOB:_NBUF + _NOB + 3]
    rsems = refs[_NBUF + _NOB + 3:_NBUF + _NOB + 3 + _NBUF]
    wsems = refs[_NBUF + _NOB + 3 + _NBUF:]

    def rcopy(t):
        return pltpu.make_async_copy(
            x_hbm.at[pl.ds(t * cb, cb)], bufs[t % _NBUF], rsems[t % _NBUF])

    def wcopy(t):
        return pltpu.make_async_copy(
            obufs[t % _NOB], y_hbm.at[pl.ds(t * cb, cb)], wsems[t % _NOB])

    for t in range(min(_NBUF, nblk)):
        rcopy(t).start()
    for t in range(nblk):
        rcopy(t).wait()
        _stats_block(bufs[t % _NBUF], sum_ref, topk_ref, t, cb)
        nt = t + _NBUF
        if nt < nblk:
            rcopy(nt).start()

    for t in range(min(_NBUF, nblk)):
        rcopy(t).start()

    avg = sum_ref[...] * inv_n
    tk = topk_ref[...]

    def fc(v):
        h = jnp.dot(w1_ref[...], v, preferred_element_type=jnp.float32)
        h = jnp.maximum(h + b1_ref[...], 0.0)
        o = jnp.dot(w2_ref[...], h, preferred_element_type=jnp.float32)
        return o + b2_ref[...]

    gate = jax.nn.sigmoid(fc(avg) + fc(tk))
    gatev_ref[...] = gate
    gate_ref[...] = gate

    for t in range(nblk):
        rcopy(t).wait()
        if t - _NOB >= 0:
            wcopy(t - _NOB).wait()
        gv = gatev_ref[pl.ds(t * cb, cb), :]
        obufs[t % _NOB][...] = bufs[t % _NBUF][...] * gv[:, :, None]
        wcopy(t).start()
        nt = t + _NBUF
        if nt < nblk:
            rcopy(nt).start()
    for t in range(max(0, nblk - _NOB), nblk):
        wcopy(t).wait()


def kernel(x, W1, b1, W2, b2):
    b, c, d, h, w = x.shape
    n = d * h * w
    assert b == 1
    lanes = 128
    rows = n // lanes
    xr = x.reshape(c, rows, lanes)

    cb = 8
    nblk = c // cb

    y, gate = pl.pallas_call(
        functools.partial(_ring_kernel, nblk=nblk, cb=cb, inv_n=1.0 / n),
        in_specs=[
            pl.BlockSpec(memory_space=pl.ANY),
            pl.BlockSpec((c // 2, c), lambda: (0, 0)),
            pl.BlockSpec((c // 2, 1), lambda: (0, 0)),
            pl.BlockSpec((c, c // 2), lambda: (0, 0)),
            pl.BlockSpec((c, 1), lambda: (0, 0)),
        ],
        out_specs=[
            pl.BlockSpec(memory_space=pl.ANY),
            pl.BlockSpec((c, 1), lambda: (0, 0)),
        ],
        out_shape=[
            jax.ShapeDtypeStruct((c, rows, lanes), jnp.float32),
            jax.ShapeDtypeStruct((c, 1), jnp.float32),
        ],
        scratch_shapes=(
            [pltpu.VMEM((cb, rows, lanes), jnp.float32)
             for _ in range(_NBUF + _NOB)]
            + [pltpu.VMEM((c, 1), jnp.float32) for _ in range(3)]
            + [pltpu.SemaphoreType.DMA for _ in range(_NBUF + _NOB)]
        ),
        compiler_params=pltpu.CompilerParams(
            vmem_limit_bytes=64 * 1024 * 1024,
        ),
    )(xr, W1, b1[:, None], W2, b2[:, None])

    out = gate.reshape(b, c, 1, 1, 1)
    return (y.reshape(b, c, d, h, w), out)
